# Initial kernel scaffold; baseline (speedup 1.0000x reference)
import jax
import jax.numpy as jnp
from jax.experimental import pallas as pl


def kernel(x, edge_index, batch, W0, b0, gw0, gb0, ga0, W1, b1, gw1, gb1, ga1, W2, b2, gw2, gb2, ga2):
    raise NotImplementedError("write your pallas kernel here")


# speedup vs baseline: 8.7558x; 8.7558x over previous
import functools

import jax
import jax.numpy as jnp
from jax import lax
from jax.experimental import pallas as pl
from jax.experimental.pallas import tpu as pltpu
from jax.experimental.pallas import tpu_sc as plsc

N = 10000
E = 320000
D = 128
G = 64
EPS = 1e-5

NPAD = 10240
DUMP = N
NC = 2
NS = 16
NW = NC * NS
CH = 128
CPT = 79
EP = NW * CPT * CH
RPT = NPAD // NS

_mesh = plsc.VectorSubcoreMesh(core_axis_name="c", subcore_axis_name="s")


@functools.partial(
    pl.kernel,
    out_type=jax.ShapeDtypeStruct((NC, NPAD, 1), jnp.float32),
    mesh=_mesh,
    scratch_types=[
        pltpu.VMEM((CPT, CH), jnp.int32),
        pltpu.VMEM((CH, 1), jnp.float32),
        pltpu.VMEM_SHARED((NPAD, 1), jnp.float32),
    ],
)
def _deg_kernel(dst_hbm, ones_hbm, zero_hbm, out_hbm, dst_v, ones_v, acc):
    cid = lax.axis_index("c")
    sid = lax.axis_index("s")
    wid = sid * NC + cid
    pltpu.sync_copy(zero_hbm, acc.at[pl.ds(sid * RPT, RPT)])
    pltpu.sync_copy(dst_hbm.at[wid], dst_v)
    pltpu.sync_copy(ones_hbm, ones_v)
    plsc.subcore_barrier()

    def body(c, carry):
        pltpu.sync_copy(ones_v, acc.at[dst_v.at[c]], add=True)
        return carry

    lax.fori_loop(0, CPT, body, 0)
    plsc.subcore_barrier()
    pltpu.sync_copy(acc.at[pl.ds(sid * RPT, RPT)],
                    out_hbm.at[cid, pl.ds(sid * RPT, RPT)])


@functools.partial(
    pl.kernel,
    out_type=jax.ShapeDtypeStruct((NC, NPAD, D), jnp.float32),
    mesh=_mesh,
    scratch_types=[
        pltpu.VMEM((CPT, CH), jnp.int32),
        pltpu.VMEM((CPT, CH), jnp.int32),
        pltpu.VMEM((CH, D), jnp.float32),
        pltpu.VMEM_SHARED((NPAD, D), jnp.float32),
        pltpu.SemaphoreType.DMA,
    ],
)
def _agg_kernel(src_hbm, dst_hbm, hp_hbm, zero_hbm, out_hbm,
                src_v, dst_v, rows_v, acc, sem):
    cid = lax.axis_index("c")
    sid = lax.axis_index("s")
    wid = sid * NC + cid
    pltpu.sync_copy(zero_hbm, acc.at[pl.ds(sid * RPT, RPT)])
    pltpu.sync_copy(src_hbm.at[wid], src_v)
    pltpu.sync_copy(dst_hbm.at[wid], dst_v)
    plsc.subcore_barrier()

    def body(c, carry):
        pltpu.async_copy(hp_hbm.at[src_v.at[c]], rows_v, sem).wait()
        pltpu.sync_copy(rows_v, acc.at[dst_v.at[c]], add=True)
        return carry

    lax.fori_loop(0, CPT, body, 0)
    plsc.subcore_barrier()
    pltpu.sync_copy(acc.at[pl.ds(sid * RPT, RPT)],
                    out_hbm.at[cid, pl.ds(sid * RPT, RPT)])


def _mm_scale_body(x_ref, w_ref, degp_ref, hp_ref, dinv_ref):
    deg = degp_ref[0] + degp_ref[1] + 1.0
    dinv = lax.rsqrt(deg)
    h = jnp.dot(x_ref[...], w_ref[...], preferred_element_type=jnp.float32)
    hp_ref[...] = h * dinv
    dinv_ref[...] = dinv


_mm_scale = pl.pallas_call(
    _mm_scale_body,
    out_shape=(
        jax.ShapeDtypeStruct((NPAD, D), jnp.float32),
        jax.ShapeDtypeStruct((NPAD, 1), jnp.float32),
    ),
)


def _seg_ops(batch_ref):
    gi = lax.broadcasted_iota(jnp.int32, (NPAD, G), 1)
    ot = (batch_ref[...] == gi).astype(jnp.float32)
    ones = jnp.ones((NPAD, 1), jnp.float32)
    cnt = jnp.maximum(
        lax.dot_general(ot, ones, (((0,), (0,)), ((), ())),
                        precision=lax.Precision.HIGHEST,
                        preferred_element_type=jnp.float32), 1.0)

    def segsum(v):
        return lax.dot_general(ot, v, (((0,), (0,)), ((), ())),
                               precision=lax.Precision.HIGHEST,
                               preferred_element_type=jnp.float32)

    def bcast(m):
        return jnp.dot(ot, m, precision=lax.Precision.HIGHEST,
                       preferred_element_type=jnp.float32)

    return cnt, segsum, bcast


def _stats_body(aggp_ref, hp_ref, dinv_ref, batch_ref, b_ref,
                t_ref, mean_ref):
    t = dinv_ref[...] * (aggp_ref[0] + aggp_ref[1] + hp_ref[...]) + b_ref[...]
    cnt, segsum, _ = _seg_ops(batch_ref)
    t_ref[...] = t
    mean_ref[...] = segsum(t) / cnt


_stats = pl.pallas_call(
    _stats_body,
    out_shape=(
        jax.ShapeDtypeStruct((NPAD, D), jnp.float32),
        jax.ShapeDtypeStruct((G, D), jnp.float32),
    ),
)


def _graph_norm(t_ref, mean_ref, batch_ref, gw_ref, gb_ref, ga_ref):
    cnt, segsum, bcast = _seg_ops(batch_ref)
    u = t_ref[...] - ga_ref[...] * bcast(mean_ref[...])
    var = segsum(u * u) / cnt
    h = gw_ref[...] * u * lax.rsqrt(bcast(var) + EPS) + gb_ref[...]
    return h, cnt, segsum


def _norm_body(t_ref, mean_ref, batch_ref, gw_ref, gb_ref, ga_ref, out_ref):
    h, _, _ = _graph_norm(t_ref, mean_ref, batch_ref, gw_ref, gb_ref, ga_ref)
    out_ref[...] = h


_norm = pl.pallas_call(
    _norm_body,
    out_shape=jax.ShapeDtypeStruct((NPAD, D), jnp.float32),
)


def _proj_body(h_ref, wn_ref, dinv_ref, out_ref):
    out_ref[...] = jnp.dot(h_ref[...], wn_ref[...],
                           preferred_element_type=jnp.float32) * dinv_ref[...]


_proj = pl.pallas_call(
    _proj_body,
    out_shape=jax.ShapeDtypeStruct((NPAD, D), jnp.float32),
)


def _final_body(t_ref, mean_ref, batch_ref,
                gw_ref, gb_ref, ga_ref, out_ref):
    h, cnt, segsum = _graph_norm(t_ref, mean_ref, batch_ref,
                                 gw_ref, gb_ref, ga_ref)
    out_ref[...] = segsum(h) / cnt


_final = pl.pallas_call(
    _final_body,
    out_shape=jax.ShapeDtypeStruct((G, D), jnp.float32),
)


def kernel(x, edge_index, batch, W0, b0, gw0, gb0, ga0,
           W1, b1, gw1, gb1, ga1, W2, b2, gw2, gb2, ga2):
    src, dst = edge_index[0], edge_index[1]
    pad_e = EP - E
    src_r = jnp.concatenate(
        [src, jnp.zeros((pad_e,), jnp.int32)]).reshape(NW, CPT, CH)
    dst_r = jnp.concatenate(
        [dst, jnp.full((pad_e,), DUMP, jnp.int32)]).reshape(NW, CPT, CH)
    x_p = jnp.pad(x, ((0, NPAD - N), (0, 0)))
    batch_col = jnp.pad(batch, (0, NPAD - N),
                        constant_values=G).reshape(NPAD, 1)
    ones_col = jnp.ones((CH, 1), jnp.float32)
    zero_col = jnp.zeros((RPT, 1), jnp.float32)
    zero_rows = jnp.zeros((RPT, D), jnp.float32)

    degp = _deg_kernel(dst_r, ones_col, zero_col)
    hp, dinv = _mm_scale(x_p, W0, degp)

    aggp = _agg_kernel(src_r, dst_r, hp, zero_rows)
    t, mean = _stats(aggp, hp, dinv, batch_col, b0)
    hp = _proj(_norm(t, mean, batch_col, gw0, gb0, ga0), W1, dinv)

    aggp = _agg_kernel(src_r, dst_r, hp, zero_rows)
    t, mean = _stats(aggp, hp, dinv, batch_col, b1)
    hp = _proj(_norm(t, mean, batch_col, gw1, gb1, ga1), W2, dinv)

    aggp = _agg_kernel(src_r, dst_r, hp, zero_rows)
    t, mean = _stats(aggp, hp, dinv, batch_col, b2)
    return _final(t, mean, batch_col, gw2, gb2, ga2)
